# baseline (speedup 1.0000x reference)
import functools

import jax
import jax.numpy as jnp
from jax import lax
from jax.experimental import pallas as pl
from jax.experimental.pallas import tpu as pltpu
from jax.experimental.pallas import tpu_sc as plsc

N = 10000
E = 320000
D = 128
G = 64
OUT = 10

NC = 2
NS = 16
NW = NC * NS

CPT = 50
EPT = E // NW
NCH = EPT // CPT
GRP = 25
NGRP = NCH // GRP
NBUF = 4
PD = NBUF - 1
NP = 10240
RPT = NP // NS

def _seg_sum_body(h_hbm, src_hbm, dst_hbm, out_hbm,
                  src_v, dst_v, rows_v, acc_sh,
                  gs0, gs1, gs2, gs3, gs4, gs5,
                  ss0, ss1, ss2, ss3, ss4, ss5, is0, is1):
    cid = lax.axis_index("c")
    sid = lax.axis_index("s")
    wid = cid * NS + sid
    gsem = (gs0, gs1, gs2, gs3, gs4, gs5)
    ssem = (ss0, ss1, ss2, ss3, ss4, ss5)
    isem = (is0, is1)

    zz = jnp.zeros((16,), jnp.float32)

    def _zrow(r, carry):
        for j in range(D // 16):
            rows_v[0, r, pl.ds(j * 16, 16)] = zz
        return carry

    lax.fori_loop(0, 32, _zrow, 0)
    for t in range(RPT // 32):
        pltpu.sync_copy(rows_v.at[0, pl.ds(0, 32)],
                        acc_sh.at[pl.ds(sid * RPT + t * 32, 32)])
    plsc.subcore_barrier()

    def _gather(c):
        slot = (c // GRP) % 2
        cp = pltpu.make_async_copy(
            h_hbm.at[src_v.at[slot, c % GRP]], rows_v.at[c % NBUF],
            gsem[c % NBUF])
        cp.start()
        return cp

    def _scatter(c):
        pltpu.async_copy(
            rows_v.at[c % NBUF], acc_sh.at[dst_v.at[(c // GRP) % 2, c % GRP]],
            ssem[c % NBUF], add=True)
        return pltpu.make_async_copy(
            rows_v.at[c % NBUF], acc_sh.at[dst_v.at[(c // GRP) % 2, c % GRP]],
            ssem[c % NBUF])

    def _idx_load(g):
        s = g % 2
        a = pltpu.make_async_copy(src_hbm.at[wid, g], src_v.at[s], isem[s])
        b = pltpu.make_async_copy(dst_hbm.at[wid, g], dst_v.at[s], isem[s])
        a.start()
        b.start()
        return a, b

    idx_loads = {0: _idx_load(0), 1: _idx_load(1)}
    for cp in idx_loads[0] + idx_loads[1]:
        cp.wait()

    gathers = {}
    scatters = {}
    for c in range(NBUF):
        gathers[c] = _gather(c)

    for c in range(NCH):
        gathers[c].wait()
        if c <= NCH - NBUF - 1:
            scatters[c] = _scatter(c)
        else:
            pltpu.sync_copy(rows_v.at[c % NBUF],
                            acc_sh.at[dst_v.at[(c // GRP) % 2, c % GRP]],
                            add=True)
        if 1 <= c <= NCH - 1 - PD:
            nxt = c + PD
            scatters[c - 1].wait()
            if c % GRP == 0 and c + GRP < NCH:
                idx_loads[c // GRP + 1] = _idx_load(c // GRP + 1)
            if nxt % GRP == 0 and nxt >= 2 * GRP:
                for cp in idx_loads[nxt // GRP]:
                    cp.wait()
            gathers[nxt] = _gather(nxt)

    plsc.subcore_barrier()
    pltpu.sync_copy(acc_sh.at[pl.ds(sid * RPT, RPT)],
                    out_hbm.at[cid, pl.ds(sid * RPT, RPT)])


@functools.cache
def _get_seg_sum():
    mesh = plsc.VectorSubcoreMesh(
        core_axis_name="c", subcore_axis_name="s",
        num_cores=NC, num_subcores=NS)
    return pl.kernel(
        _seg_sum_body,
        out_type=jax.ShapeDtypeStruct((NC, NP, D), jnp.float32),
        mesh=mesh,
        scratch_types=[
            pltpu.VMEM((2, GRP, CPT), jnp.int32),
            pltpu.VMEM((2, GRP, CPT), jnp.int32),
            pltpu.VMEM((NBUF, CPT, D), jnp.float32),
            pltpu.VMEM_SHARED((NP, D), jnp.float32),
            pltpu.SemaphoreType.DMA,
            pltpu.SemaphoreType.DMA,
            pltpu.SemaphoreType.DMA,
            pltpu.SemaphoreType.DMA,
            pltpu.SemaphoreType.DMA,
            pltpu.SemaphoreType.DMA,
            pltpu.SemaphoreType.DMA,
            pltpu.SemaphoreType.DMA,
            pltpu.SemaphoreType.DMA,
            pltpu.SemaphoreType.DMA,
            pltpu.SemaphoreType.DMA,
            pltpu.SemaphoreType.DMA,
            pltpu.SemaphoreType.DMA,
            pltpu.SemaphoreType.DMA,
        ],
    )


def _seg_sum(h, src3, dst3):
    return _get_seg_sum()(h, src3, dst3)


BN = 1000
NB = N // BN


def _layer_body(acc_ref, h_ref, batch_ref, wr_ref, br_ref, wo_ref,
                hn_ref, ps_ref, pm_ref, cnt_ref):
    i = pl.program_id(0)
    agg = acc_ref[0] + acc_ref[1]
    hn = jnp.dot(agg, wr_ref[...], preferred_element_type=jnp.float32)
    hn = hn + jnp.dot(h_ref[...], wo_ref[...], preferred_element_type=jnp.float32)
    hn = jnp.maximum(hn + br_ref[...], 0.0)
    hn_ref[...] = hn

    bb = batch_ref[...][:, 0]

    @pl.when(i == 0)
    def _():
        ps_ref[...] = jnp.zeros((G, D), jnp.float32)
        pm_ref[...] = jnp.zeros((G, D), jnp.float32)
        cnt_ref[...] = jnp.zeros((G, D), jnp.float32)

    lo = bb[0]
    hi = bb[BN - 1]
    gcol = lax.broadcasted_iota(jnp.int32, (G, 1), 0)

    def _g(g, carry):
        cur_s, cur_m, cur_c = carry
        mrow = (bb == g)[:, None]
        hm = jnp.where(mrow, hn, 0.0)
        s_g = jnp.sum(hm, axis=0)
        m_g = jnp.max(hm, axis=0)
        c_g = jnp.sum(mrow.astype(jnp.float32))
        sel = gcol == g
        cur_s = cur_s + jnp.where(sel, s_g[None, :], 0.0)
        cur_m = jnp.maximum(cur_m, jnp.where(sel, m_g[None, :], 0.0))
        cur_c = cur_c + jnp.where(sel, c_g, 0.0)
        return cur_s, cur_m, cur_c

    s0, m0, c0 = lax.fori_loop(
        lo, hi + 1, _g, (ps_ref[...], pm_ref[...], cnt_ref[...]))
    ps_ref[...] = s0
    pm_ref[...] = m0
    cnt_ref[...] = c0


def _layer_call(acc, h, batch2d, wr, br, wo):
    return pl.pallas_call(
        _layer_body,
        grid=(NB,),
        in_specs=[
            pl.BlockSpec((NC, BN, D), lambda i: (0, i, 0)),
            pl.BlockSpec((BN, D), lambda i: (i, 0)),
            pl.BlockSpec((BN, 1), lambda i: (i, 0)),
            pl.BlockSpec((D, D), lambda i: (0, 0)),
            pl.BlockSpec((1, D), lambda i: (0, 0)),
            pl.BlockSpec((D, D), lambda i: (0, 0)),
        ],
        out_specs=[
            pl.BlockSpec((BN, D), lambda i: (i, 0)),
            pl.BlockSpec((G, D), lambda i: (0, 0)),
            pl.BlockSpec((G, D), lambda i: (0, 0)),
            pl.BlockSpec((G, D), lambda i: (0, 0)),
        ],
        out_shape=[
            jax.ShapeDtypeStruct((N, D), jnp.float32),
            jax.ShapeDtypeStruct((G, D), jnp.float32),
            jax.ShapeDtypeStruct((G, D), jnp.float32),
            jax.ShapeDtypeStruct((G, D), jnp.float32),
        ],
    )(acc, h, batch2d, wr, br, wo)


def _head_body(s1, s2, s3, m1, m2, m3, cnt,
               w1, b1, w2, b2, w3, b3, out_ref):
    S = s1[...] + s2[...] + s3[...]
    M = m1[...] + m2[...] + m3[...]
    mean = S / jnp.maximum(cnt[...], 1.0)
    z = jnp.concatenate([mean, M], axis=1)
    z = jnp.maximum(
        jnp.dot(z, w1[...], preferred_element_type=jnp.float32) + b1[...], 0.0)
    z = jnp.maximum(
        jnp.dot(z, w2[...], preferred_element_type=jnp.float32) + b2[...], 0.0)
    out_ref[...] = (
        jnp.dot(z, w3[...], preferred_element_type=jnp.float32) + b3[...])


def _head_call(s1, s2, s3, m1, m2, m3, cnt, w1, b1, w2, b2, w3, b3):
    return pl.pallas_call(
        _head_body,
        out_shape=jax.ShapeDtypeStruct((G, OUT), jnp.float32),
    )(s1, s2, s3, m1, m2, m3, cnt, w1, b1, w2, b2, w3, b3)


def kernel(x, edge_index, batch,
           W_rel0, b_rel0, W_root0,
           W_rel1, b_rel1, W_root1,
           W_rel2, b_rel2, W_root2,
           lin1_W, lin1_b, lin2_W, lin2_b, lin3_W, lin3_b):
    src3 = edge_index[0].reshape(NW, NGRP, GRP, CPT)
    dst3 = edge_index[1].reshape(NW, NGRP, GRP, CPT)
    batch2d = batch.reshape(N, 1)

    acc = _seg_sum(x, src3, dst3)
    h1, s1, m1, cnt = _layer_call(acc, x, batch2d,
                                  W_rel0, b_rel0.reshape(1, D), W_root0)
    acc = _seg_sum(h1, src3, dst3)
    h2, s2, m2, _ = _layer_call(acc, h1, batch2d,
                                W_rel1, b_rel1.reshape(1, D), W_root1)
    acc = _seg_sum(h2, src3, dst3)
    h3, s3, m3, _ = _layer_call(acc, h2, batch2d,
                                W_rel2, b_rel2.reshape(1, D), W_root2)

    return _head_call(s1, s2, s3, m1, m2, m3, cnt,
                      lin1_W, lin1_b.reshape(1, G), lin2_W,
                      lin2_b.reshape(1, 32), lin3_W, lin3_b.reshape(1, OUT))


# speedup vs baseline: 1.0520x; 1.0520x over previous
import functools

import jax
import jax.numpy as jnp
from jax import lax
from jax.experimental import pallas as pl
from jax.experimental.pallas import tpu as pltpu
from jax.experimental.pallas import tpu_sc as plsc

N = 10000
E = 320000
D = 128
G = 64
OUT = 10

NC = 2
NS = 16
NW = NC * NS

CPT = 40
EPT = E // NW
NCH = EPT // CPT
GRP = 25
NGRP = NCH // GRP
NBUF = 6
PD = NBUF - 1
NP = 10240
RPT = NP // NS

def _seg_sum_body(h_hbm, src_hbm, dst_hbm, out_hbm,
                  src_v, dst_v, rows_v, acc_sh,
                  gs0, gs1, gs2, gs3, gs4, gs5,
                  ss0, ss1, ss2, ss3, ss4, ss5, is0, is1):
    cid = lax.axis_index("c")
    sid = lax.axis_index("s")
    wid = cid * NS + sid
    gsem = (gs0, gs1, gs2, gs3, gs4, gs5)
    ssem = (ss0, ss1, ss2, ss3, ss4, ss5)
    isem = (is0, is1)

    zz = jnp.zeros((16,), jnp.float32)

    def _zrow(r, carry):
        for j in range(D // 16):
            rows_v[0, r, pl.ds(j * 16, 16)] = zz
        return carry

    lax.fori_loop(0, CPT, _zrow, 0)
    for t in range(RPT // CPT):
        pltpu.sync_copy(rows_v.at[0],
                        acc_sh.at[pl.ds(sid * RPT + t * CPT, CPT)])
    plsc.subcore_barrier()

    def _gather(c):
        slot = (c // GRP) % 2
        cp = pltpu.make_async_copy(
            h_hbm.at[src_v.at[slot, c % GRP]], rows_v.at[c % NBUF],
            gsem[c % NBUF])
        cp.start()
        return cp

    def _scatter(c):
        pltpu.async_copy(
            rows_v.at[c % NBUF], acc_sh.at[dst_v.at[(c // GRP) % 2, c % GRP]],
            ssem[c % NBUF], add=True)
        return pltpu.make_async_copy(
            rows_v.at[c % NBUF], acc_sh.at[dst_v.at[(c // GRP) % 2, c % GRP]],
            ssem[c % NBUF])

    def _idx_load(g):
        s = g % 2
        a = pltpu.make_async_copy(src_hbm.at[wid, g], src_v.at[s], isem[s])
        b = pltpu.make_async_copy(dst_hbm.at[wid, g], dst_v.at[s], isem[s])
        a.start()
        b.start()
        return a, b

    idx_loads = {0: _idx_load(0), 1: _idx_load(1)}
    for cp in idx_loads[0] + idx_loads[1]:
        cp.wait()

    gathers = {}
    scatters = {}
    for c in range(NBUF):
        gathers[c] = _gather(c)

    for c in range(NCH):
        gathers[c].wait()
        if c <= NCH - NBUF - 1:
            scatters[c] = _scatter(c)
        else:
            pltpu.sync_copy(rows_v.at[c % NBUF],
                            acc_sh.at[dst_v.at[(c // GRP) % 2, c % GRP]],
                            add=True)
        if 1 <= c <= NCH - 1 - PD:
            nxt = c + PD
            scatters[c - 1].wait()
            if c % GRP == 0 and c + GRP < NCH:
                idx_loads[c // GRP + 1] = _idx_load(c // GRP + 1)
            if nxt % GRP == 0 and nxt >= 2 * GRP:
                for cp in idx_loads[nxt // GRP]:
                    cp.wait()
            gathers[nxt] = _gather(nxt)

    plsc.subcore_barrier()
    pltpu.sync_copy(acc_sh.at[pl.ds(sid * RPT, RPT)],
                    out_hbm.at[cid, pl.ds(sid * RPT, RPT)])


@functools.cache
def _get_seg_sum():
    mesh = plsc.VectorSubcoreMesh(
        core_axis_name="c", subcore_axis_name="s",
        num_cores=NC, num_subcores=NS)
    return pl.kernel(
        _seg_sum_body,
        out_type=jax.ShapeDtypeStruct((NC, NP, D), jnp.float32),
        mesh=mesh,
        scratch_types=[
            pltpu.VMEM((2, GRP, CPT), jnp.int32),
            pltpu.VMEM((2, GRP, CPT), jnp.int32),
            pltpu.VMEM((NBUF, CPT, D), jnp.float32),
            pltpu.VMEM_SHARED((NP, D), jnp.float32),
            pltpu.SemaphoreType.DMA,
            pltpu.SemaphoreType.DMA,
            pltpu.SemaphoreType.DMA,
            pltpu.SemaphoreType.DMA,
            pltpu.SemaphoreType.DMA,
            pltpu.SemaphoreType.DMA,
            pltpu.SemaphoreType.DMA,
            pltpu.SemaphoreType.DMA,
            pltpu.SemaphoreType.DMA,
            pltpu.SemaphoreType.DMA,
            pltpu.SemaphoreType.DMA,
            pltpu.SemaphoreType.DMA,
            pltpu.SemaphoreType.DMA,
            pltpu.SemaphoreType.DMA,
        ],
    )


def _seg_sum(h, src3, dst3):
    return _get_seg_sum()(h, src3, dst3)


BN = 1000
NB = N // BN


def _layer_body(acc_ref, h_ref, batch_ref, wr_ref, br_ref, wo_ref,
                hn_ref, ps_ref, pm_ref, cnt_ref):
    i = pl.program_id(0)
    agg = acc_ref[0] + acc_ref[1]
    hn = jnp.dot(agg, wr_ref[...], preferred_element_type=jnp.float32)
    hn = hn + jnp.dot(h_ref[...], wo_ref[...], preferred_element_type=jnp.float32)
    hn = jnp.maximum(hn + br_ref[...], 0.0)
    hn_ref[...] = hn

    bb = batch_ref[...][:, 0]

    @pl.when(i == 0)
    def _():
        ps_ref[...] = jnp.zeros((G, D), jnp.float32)
        pm_ref[...] = jnp.zeros((G, D), jnp.float32)
        cnt_ref[...] = jnp.zeros((G, D), jnp.float32)

    lo = bb[0]
    hi = bb[BN - 1]
    gcol = lax.broadcasted_iota(jnp.int32, (G, 1), 0)

    def _g(g, carry):
        cur_s, cur_m, cur_c = carry
        mrow = (bb == g)[:, None]
        hm = jnp.where(mrow, hn, 0.0)
        s_g = jnp.sum(hm, axis=0)
        m_g = jnp.max(hm, axis=0)
        c_g = jnp.sum(mrow.astype(jnp.float32))
        sel = gcol == g
        cur_s = cur_s + jnp.where(sel, s_g[None, :], 0.0)
        cur_m = jnp.maximum(cur_m, jnp.where(sel, m_g[None, :], 0.0))
        cur_c = cur_c + jnp.where(sel, c_g, 0.0)
        return cur_s, cur_m, cur_c

    s0, m0, c0 = lax.fori_loop(
        lo, hi + 1, _g, (ps_ref[...], pm_ref[...], cnt_ref[...]))
    ps_ref[...] = s0
    pm_ref[...] = m0
    cnt_ref[...] = c0


def _layer_call(acc, h, batch2d, wr, br, wo):
    return pl.pallas_call(
        _layer_body,
        grid=(NB,),
        in_specs=[
            pl.BlockSpec((NC, BN, D), lambda i: (0, i, 0)),
            pl.BlockSpec((BN, D), lambda i: (i, 0)),
            pl.BlockSpec((BN, 1), lambda i: (i, 0)),
            pl.BlockSpec((D, D), lambda i: (0, 0)),
            pl.BlockSpec((1, D), lambda i: (0, 0)),
            pl.BlockSpec((D, D), lambda i: (0, 0)),
        ],
        out_specs=[
            pl.BlockSpec((BN, D), lambda i: (i, 0)),
            pl.BlockSpec((G, D), lambda i: (0, 0)),
            pl.BlockSpec((G, D), lambda i: (0, 0)),
            pl.BlockSpec((G, D), lambda i: (0, 0)),
        ],
        out_shape=[
            jax.ShapeDtypeStruct((N, D), jnp.float32),
            jax.ShapeDtypeStruct((G, D), jnp.float32),
            jax.ShapeDtypeStruct((G, D), jnp.float32),
            jax.ShapeDtypeStruct((G, D), jnp.float32),
        ],
    )(acc, h, batch2d, wr, br, wo)


def _head_body(s1, s2, s3, m1, m2, m3, cnt,
               w1, b1, w2, b2, w3, b3, out_ref):
    S = s1[...] + s2[...] + s3[...]
    M = m1[...] + m2[...] + m3[...]
    mean = S / jnp.maximum(cnt[...], 1.0)
    z = jnp.concatenate([mean, M], axis=1)
    z = jnp.maximum(
        jnp.dot(z, w1[...], preferred_element_type=jnp.float32) + b1[...], 0.0)
    z = jnp.maximum(
        jnp.dot(z, w2[...], preferred_element_type=jnp.float32) + b2[...], 0.0)
    out_ref[...] = (
        jnp.dot(z, w3[...], preferred_element_type=jnp.float32) + b3[...])


def _head_call(s1, s2, s3, m1, m2, m3, cnt, w1, b1, w2, b2, w3, b3):
    return pl.pallas_call(
        _head_body,
        out_shape=jax.ShapeDtypeStruct((G, OUT), jnp.float32),
    )(s1, s2, s3, m1, m2, m3, cnt, w1, b1, w2, b2, w3, b3)


def kernel(x, edge_index, batch,
           W_rel0, b_rel0, W_root0,
           W_rel1, b_rel1, W_root1,
           W_rel2, b_rel2, W_root2,
           lin1_W, lin1_b, lin2_W, lin2_b, lin3_W, lin3_b):
    src3 = edge_index[0].reshape(NW, NGRP, GRP, CPT)
    dst3 = edge_index[1].reshape(NW, NGRP, GRP, CPT)
    batch2d = batch.reshape(N, 1)

    acc = _seg_sum(x, src3, dst3)
    h1, s1, m1, cnt = _layer_call(acc, x, batch2d,
                                  W_rel0, b_rel0.reshape(1, D), W_root0)
    acc = _seg_sum(h1, src3, dst3)
    h2, s2, m2, _ = _layer_call(acc, h1, batch2d,
                                W_rel1, b_rel1.reshape(1, D), W_root1)
    acc = _seg_sum(h2, src3, dst3)
    h3, s3, m3, _ = _layer_call(acc, h2, batch2d,
                                W_rel2, b_rel2.reshape(1, D), W_root2)

    return _head_call(s1, s2, s3, m1, m2, m3, cnt,
                      lin1_W, lin1_b.reshape(1, G), lin2_W,
                      lin2_b.reshape(1, 32), lin3_W, lin3_b.reshape(1, OUT))
